# baseline (speedup 1.0000x reference)
import functools

import jax
import jax.numpy as jnp
from jax import lax
from jax.experimental import pallas as pl
from jax.experimental.pallas import tpu as pltpu
from jax.experimental.pallas import tpu_sc as plsc

_B, _N = 1024, 32768
_MARGIN = 0.1
_TOPK = 10
_BB = 8

_NC, _NS = 2, 16
_NW = _NC * _NS
_RPW = _B // _NW
_D = 128
_SEG = _N // _D


def _select_body(tgt_ref, simf_ref, noise_ref, idx_ref):
    tgt = tgt_ref[...]
    simf = simf_ref[...]
    noise = noise_ref[...]

    neg_inf = jnp.float32(-jnp.inf)
    col = jax.lax.broadcasted_iota(jnp.int32, (_BB, _N), 1)
    lanek = jax.lax.broadcasted_iota(jnp.int32, (_BB, 16), 1)
    acc = jnp.zeros((_BB, 16), jnp.int32)

    ub = lax.shift_right_logical(noise, 9)
    z = jnp.where(tgt > 0, ub, jnp.int32(-1))
    zmax = jnp.max(z, axis=1)
    jp = jnp.min(jnp.where(z == zmax[:, None], col, jnp.int32(_N)), axis=1)
    acc = jnp.where(lanek == _TOPK, jp[:, None], acc)

    s = jnp.where(tgt == 0, simf, jnp.float32(-50.0))
    m = jnp.max(s, axis=1)
    jn = jnp.min(jnp.where(s == m[:, None], col, jnp.int32(_N)), axis=1)
    acc = jnp.where(lanek == 0, jn[:, None], acc)
    for k in range(1, _TOPK):
        mb = m[:, None]
        jb = jn[:, None]
        elig = (s < mb) | ((s == mb) & (col > jb))
        m2 = jnp.max(jnp.where(elig, s, neg_inf), axis=1)
        m2b = m2[:, None]
        jn = jnp.min(
            jnp.where((s == m2b) & ((m2b < mb) | (col > jb)), col,
                      jnp.int32(_N)),
            axis=1)
        m = m2
        acc = jnp.where(lanek == k, jn[:, None], acc)

    idx_ref[...] = acc


_sc_mesh = plsc.VectorSubcoreMesh(core_axis_name="c", subcore_axis_name="s")


@functools.partial(
    pl.kernel,
    mesh=_sc_mesh,
    out_type=jax.ShapeDtypeStruct((_NW * 64,), jnp.float32),
    scratch_types=[
        pltpu.VMEM((_RPW,), jnp.int32),
        pltpu.VMEM((_RPW * _TOPK,), jnp.int32),
        pltpu.VMEM((_RPW,), jnp.float32),
        pltpu.VMEM((_RPW,), jnp.float32),
        pltpu.VMEM((_RPW * _TOPK,), jnp.float32),
        pltpu.VMEM((_RPW * _TOPK,), jnp.float32),
        pltpu.VMEM((64,), jnp.float32),
        pltpu.SemaphoreType.DMA,
    ],
)
def _gather_loss(ti_hbm, tf_hbm, pidx_hbm, nidx_hbm, out_hbm, pidx_v, nidx_v,
                 pi_v, pf_v, ni_v, nf_v, acc_v, sem):
    wid = lax.axis_index("s") * _NC + lax.axis_index("c")
    pbase = wid * _RPW
    nbase = wid * (_RPW * _TOPK)
    pltpu.sync_copy(pidx_hbm.at[pl.ds(pbase, _RPW)], pidx_v)
    pltpu.sync_copy(nidx_hbm.at[pl.ds(nbase, _RPW * _TOPK)], nidx_v)
    pltpu.async_copy(ti_hbm.at[pidx_v], pi_v, sem).wait()
    pltpu.async_copy(tf_hbm.at[pidx_v], pf_v, sem).wait()
    pltpu.async_copy(ti_hbm.at[nidx_v], ni_v, sem).wait()
    pltpu.async_copy(tf_hbm.at[nidx_v], nf_v, sem).wait()

    margin = jnp.full((16,), _MARGIN, jnp.float32)
    zero = jnp.zeros((16,), jnp.float32)
    one = jnp.ones((16,), jnp.float32)

    sp = zero
    cp = zero
    for c in range(_RPW // 16):
        si = pi_v[pl.ds(c * 16, 16)]
        sf = pf_v[pl.ds(c * 16, 16)]
        t = jnp.maximum(si - sf + margin, zero)
        sp = sp + t
        cp = cp + jnp.where(t > zero, one, zero)

    sn = zero
    cn = zero
    for c in range(_RPW * _TOPK // 16):
        si = ni_v[pl.ds(c * 16, 16)]
        sf = nf_v[pl.ds(c * 16, 16)]
        t = jnp.maximum(sf - si + margin, zero)
        sn = sn + t
        cn = cn + jnp.where(t > zero, one, zero)

    acc_v[pl.ds(0, 16)] = sp
    acc_v[pl.ds(16, 16)] = cp
    acc_v[pl.ds(32, 16)] = sn
    acc_v[pl.ds(48, 16)] = cn
    pltpu.sync_copy(acc_v, out_hbm.at[pl.ds(wid * 64, 64)])


def _combine_body(p_ref, out_ref):
    p = p_ref[...]
    kind = jax.lax.broadcasted_iota(jnp.int32, (_NW * 4, 16), 0) % 4
    zero = jnp.float32(0.0)
    sp = jnp.sum(jnp.where(kind == 0, p, zero))
    cp = jnp.sum(jnp.where(kind == 1, p, zero))
    sn = jnp.sum(jnp.where(kind == 2, p, zero))
    cn = jnp.sum(jnp.where(kind == 3, p, zero))
    lp = jnp.where(sp == zero, zero, sp / jnp.maximum(cp, 1.0))
    ln = jnp.where(sn == zero, zero, sn / jnp.maximum(cn, 1.0))
    out_ref[...] = ((lp + ln) * 0.5).reshape(1, 1)


def kernel(sim_i, sim_f, target):
    noise = jax.lax.bitcast_convert_type(
        jax.random.bits(jax.random.key(42), (_B, _N), jnp.uint32), jnp.int32)

    spec = pl.BlockSpec((_BB, _N), lambda i: (i, 0))
    idx = pl.pallas_call(
        _select_body,
        grid=(_B // _BB,),
        in_specs=[spec, spec, spec],
        out_specs=pl.BlockSpec((_BB, 16), lambda i: (i, 0)),
        out_shape=jax.ShapeDtypeStruct((_B, 16), jnp.int32),
    )(target, sim_f, noise)

    jn = idx[:, :_TOPK]
    jp = idx[:, _TOPK]
    rows = jnp.arange(_B, dtype=jnp.int32)
    pos_flat = rows * _N + jp
    neg_flat = (rows[:, None] * _N + jn).reshape(-1)

    partials = _gather_loss(
        sim_i.reshape(_B * _N),
        sim_f.reshape(_B * _N),
        pos_flat, neg_flat,
    )

    out = pl.pallas_call(
        _combine_body,
        out_shape=jax.ShapeDtypeStruct((1, 1), jnp.float32),
    )(partials.reshape(_NW * 4, 16))
    return out[0, 0]


# speedup vs baseline: 1.1691x; 1.1691x over previous
import functools

import jax
import jax.numpy as jnp
from jax import lax
from jax.experimental import pallas as pl
from jax.experimental.pallas import tpu as pltpu
from jax.experimental.pallas import tpu_sc as plsc

_B, _N = 1024, 32768
_MARGIN = 0.1
_TOPK = 10
_BB = 8

_NC, _NS = 2, 16
_NW = _NC * _NS
_RPW = _B // _NW
_D = 128
_SEG = _N // _D


def _select_body(tgt_ref, simf_ref, noise_ref, idx_ref):
    tgt = tgt_ref[...]
    simf = simf_ref[...]
    noise = noise_ref[...]

    neg_inf = jnp.float32(-jnp.inf)
    col = jax.lax.broadcasted_iota(jnp.int32, (_BB, _N), 1)
    lanek = jax.lax.broadcasted_iota(jnp.int32, (_BB, 16), 1)
    acc = jnp.zeros((_BB, 16), jnp.int32)

    ub = lax.shift_right_logical(noise, 9)
    z = jnp.where(tgt > 0, ub, jnp.int32(-1))
    zmax = jnp.max(z, axis=1)
    jp = jnp.min(jnp.where(z == zmax[:, None], col, jnp.int32(_N)), axis=1)
    acc = jnp.where(lanek == _TOPK, jp[:, None], acc)

    s = jnp.where(tgt == 0, simf, jnp.float32(-50.0))
    for k in range(_TOPK):
        m = jnp.max(s, axis=1)
        jn = jnp.min(jnp.where(s == m[:, None], col, jnp.int32(_N)), axis=1)
        acc = jnp.where(lanek == k, jn[:, None], acc)
        s = jnp.where(col == jn[:, None], neg_inf, s)

    idx_ref[...] = acc


_sc_mesh = plsc.VectorSubcoreMesh(core_axis_name="c", subcore_axis_name="s")


@functools.partial(
    pl.kernel,
    mesh=_sc_mesh,
    out_type=jax.ShapeDtypeStruct((_NW * 64,), jnp.float32),
    scratch_types=[
        pltpu.VMEM((_RPW,), jnp.int32),
        pltpu.VMEM((_RPW * _TOPK,), jnp.int32),
        pltpu.VMEM((_RPW,), jnp.float32),
        pltpu.VMEM((_RPW,), jnp.float32),
        pltpu.VMEM((_RPW * _TOPK,), jnp.float32),
        pltpu.VMEM((_RPW * _TOPK,), jnp.float32),
        pltpu.VMEM((64,), jnp.float32),
        pltpu.SemaphoreType.DMA,
    ],
)
def _gather_loss(ti_hbm, tf_hbm, pidx_hbm, nidx_hbm, out_hbm, pidx_v, nidx_v,
                 pi_v, pf_v, ni_v, nf_v, acc_v, sem):
    wid = lax.axis_index("s") * _NC + lax.axis_index("c")
    pbase = wid * _RPW
    nbase = wid * (_RPW * _TOPK)
    pltpu.sync_copy(pidx_hbm.at[pl.ds(pbase, _RPW)], pidx_v)
    pltpu.sync_copy(nidx_hbm.at[pl.ds(nbase, _RPW * _TOPK)], nidx_v)
    pltpu.async_copy(ti_hbm.at[pidx_v], pi_v, sem).wait()
    pltpu.async_copy(tf_hbm.at[pidx_v], pf_v, sem).wait()
    pltpu.async_copy(ti_hbm.at[nidx_v], ni_v, sem).wait()
    pltpu.async_copy(tf_hbm.at[nidx_v], nf_v, sem).wait()

    margin = jnp.full((16,), _MARGIN, jnp.float32)
    zero = jnp.zeros((16,), jnp.float32)
    one = jnp.ones((16,), jnp.float32)

    sp = zero
    cp = zero
    for c in range(_RPW // 16):
        si = pi_v[pl.ds(c * 16, 16)]
        sf = pf_v[pl.ds(c * 16, 16)]
        t = jnp.maximum(si - sf + margin, zero)
        sp = sp + t
        cp = cp + jnp.where(t > zero, one, zero)

    sn = zero
    cn = zero
    for c in range(_RPW * _TOPK // 16):
        si = ni_v[pl.ds(c * 16, 16)]
        sf = nf_v[pl.ds(c * 16, 16)]
        t = jnp.maximum(sf - si + margin, zero)
        sn = sn + t
        cn = cn + jnp.where(t > zero, one, zero)

    acc_v[pl.ds(0, 16)] = sp
    acc_v[pl.ds(16, 16)] = cp
    acc_v[pl.ds(32, 16)] = sn
    acc_v[pl.ds(48, 16)] = cn
    pltpu.sync_copy(acc_v, out_hbm.at[pl.ds(wid * 64, 64)])


def _combine_body(p_ref, out_ref):
    p = p_ref[...]
    kind = jax.lax.broadcasted_iota(jnp.int32, (_NW * 4, 16), 0) % 4
    zero = jnp.float32(0.0)
    sp = jnp.sum(jnp.where(kind == 0, p, zero))
    cp = jnp.sum(jnp.where(kind == 1, p, zero))
    sn = jnp.sum(jnp.where(kind == 2, p, zero))
    cn = jnp.sum(jnp.where(kind == 3, p, zero))
    lp = jnp.where(sp == zero, zero, sp / jnp.maximum(cp, 1.0))
    ln = jnp.where(sn == zero, zero, sn / jnp.maximum(cn, 1.0))
    out_ref[...] = ((lp + ln) * 0.5).reshape(1, 1)


def kernel(sim_i, sim_f, target):
    noise = jax.lax.bitcast_convert_type(
        jax.random.bits(jax.random.key(42), (_B, _N), jnp.uint32), jnp.int32)

    spec = pl.BlockSpec((_BB, _N), lambda i: (i, 0))
    idx = pl.pallas_call(
        _select_body,
        grid=(_B // _BB,),
        in_specs=[spec, spec, spec],
        out_specs=pl.BlockSpec((_BB, 16), lambda i: (i, 0)),
        out_shape=jax.ShapeDtypeStruct((_B, 16), jnp.int32),
    )(target, sim_f, noise)

    jn = idx[:, :_TOPK]
    jp = idx[:, _TOPK]
    rows = jnp.arange(_B, dtype=jnp.int32)
    pos_flat = rows * _N + jp
    neg_flat = (rows[:, None] * _N + jn).reshape(-1)

    partials = _gather_loss(
        sim_i.reshape(_B * _N),
        sim_f.reshape(_B * _N),
        pos_flat, neg_flat,
    )

    out = pl.pallas_call(
        _combine_body,
        out_shape=jax.ShapeDtypeStruct((1, 1), jnp.float32),
    )(partials.reshape(_NW * 4, 16))
    return out[0, 0]


# speedup vs baseline: 1.6291x; 1.3935x over previous
import functools

import jax
import jax.numpy as jnp
from jax import lax
from jax.experimental import pallas as pl
from jax.experimental.pallas import tpu as pltpu
from jax.experimental.pallas import tpu_sc as plsc

_B, _N = 1024, 32768
_MARGIN = 0.1
_TOPK = 10
_BB = 16

_NC, _NS = 2, 16
_NW = _NC * _NS
_RPW = _B // _NW
_D = 128
_SEG = _N // _D


def _select_body(tgt_ref, simf_ref, noise_ref, idx_ref):
    tgt = tgt_ref[...]
    simf = simf_ref[...]
    noise = noise_ref[...]

    neg_inf = jnp.float32(-jnp.inf)
    col = jax.lax.broadcasted_iota(jnp.int32, (_BB, _N), 1)
    lanek = jax.lax.broadcasted_iota(jnp.int32, (_BB, 16), 1)
    acc = jnp.zeros((_BB, 16), jnp.int32)

    ub = lax.shift_right_logical(noise, 9)
    z = jnp.where(tgt > 0, ub, jnp.int32(-1))
    zmax = jnp.max(z, axis=1)
    jp = jnp.min(jnp.where(z == zmax[:, None], col, jnp.int32(_N)), axis=1)
    acc = jnp.where(lanek == _TOPK, jp[:, None], acc)

    s = jnp.where(tgt == 0, simf, jnp.float32(-50.0))
    for k in range(_TOPK):
        m = jnp.max(s, axis=1)
        jn = jnp.min(jnp.where(s == m[:, None], col, jnp.int32(_N)), axis=1)
        acc = jnp.where(lanek == k, jn[:, None], acc)
        s = jnp.where(col == jn[:, None], neg_inf, s)

    idx_ref[...] = acc


_sc_mesh = plsc.VectorSubcoreMesh(core_axis_name="c", subcore_axis_name="s")


@functools.partial(
    pl.kernel,
    mesh=_sc_mesh,
    out_type=jax.ShapeDtypeStruct((_NW * 64,), jnp.float32),
    scratch_types=[
        pltpu.VMEM((_RPW,), jnp.int32),
        pltpu.VMEM((_RPW * _TOPK,), jnp.int32),
        pltpu.VMEM((_RPW,), jnp.float32),
        pltpu.VMEM((_RPW,), jnp.float32),
        pltpu.VMEM((_RPW * _TOPK,), jnp.float32),
        pltpu.VMEM((_RPW * _TOPK,), jnp.float32),
        pltpu.VMEM((64,), jnp.float32),
        pltpu.SemaphoreType.DMA,
    ],
)
def _gather_loss(ti_hbm, tf_hbm, pidx_hbm, nidx_hbm, out_hbm, pidx_v, nidx_v,
                 pi_v, pf_v, ni_v, nf_v, acc_v, sem):
    wid = lax.axis_index("s") * _NC + lax.axis_index("c")
    pbase = wid * _RPW
    nbase = wid * (_RPW * _TOPK)
    pltpu.sync_copy(pidx_hbm.at[pl.ds(pbase, _RPW)], pidx_v)
    pltpu.sync_copy(nidx_hbm.at[pl.ds(nbase, _RPW * _TOPK)], nidx_v)
    pltpu.async_copy(ti_hbm.at[pidx_v], pi_v, sem).wait()
    pltpu.async_copy(tf_hbm.at[pidx_v], pf_v, sem).wait()
    pltpu.async_copy(ti_hbm.at[nidx_v], ni_v, sem).wait()
    pltpu.async_copy(tf_hbm.at[nidx_v], nf_v, sem).wait()

    margin = jnp.full((16,), _MARGIN, jnp.float32)
    zero = jnp.zeros((16,), jnp.float32)
    one = jnp.ones((16,), jnp.float32)

    sp = zero
    cp = zero
    for c in range(_RPW // 16):
        si = pi_v[pl.ds(c * 16, 16)]
        sf = pf_v[pl.ds(c * 16, 16)]
        t = jnp.maximum(si - sf + margin, zero)
        sp = sp + t
        cp = cp + jnp.where(t > zero, one, zero)

    sn = zero
    cn = zero
    for c in range(_RPW * _TOPK // 16):
        si = ni_v[pl.ds(c * 16, 16)]
        sf = nf_v[pl.ds(c * 16, 16)]
        t = jnp.maximum(sf - si + margin, zero)
        sn = sn + t
        cn = cn + jnp.where(t > zero, one, zero)

    acc_v[pl.ds(0, 16)] = sp
    acc_v[pl.ds(16, 16)] = cp
    acc_v[pl.ds(32, 16)] = sn
    acc_v[pl.ds(48, 16)] = cn
    pltpu.sync_copy(acc_v, out_hbm.at[pl.ds(wid * 64, 64)])


def _combine_body(p_ref, out_ref):
    p = p_ref[...]
    kind = jax.lax.broadcasted_iota(jnp.int32, (_NW * 4, 16), 0) % 4
    zero = jnp.float32(0.0)
    sp = jnp.sum(jnp.where(kind == 0, p, zero))
    cp = jnp.sum(jnp.where(kind == 1, p, zero))
    sn = jnp.sum(jnp.where(kind == 2, p, zero))
    cn = jnp.sum(jnp.where(kind == 3, p, zero))
    lp = jnp.where(sp == zero, zero, sp / jnp.maximum(cp, 1.0))
    ln = jnp.where(sn == zero, zero, sn / jnp.maximum(cn, 1.0))
    out_ref[...] = ((lp + ln) * 0.5).reshape(1, 1)


def kernel(sim_i, sim_f, target):
    noise = jax.lax.bitcast_convert_type(
        jax.random.bits(jax.random.key(42), (_B, _N), jnp.uint32), jnp.int32)

    spec = pl.BlockSpec((_BB, _N), lambda i: (i, 0))
    idx = pl.pallas_call(
        _select_body,
        grid=(_B // _BB,),
        in_specs=[spec, spec, spec],
        out_specs=pl.BlockSpec((_BB, 16), lambda i: (i, 0)),
        out_shape=jax.ShapeDtypeStruct((_B, 16), jnp.int32),
    )(target, sim_f, noise)

    jn = idx[:, :_TOPK]
    jp = idx[:, _TOPK]
    rows = jnp.arange(_B, dtype=jnp.int32)
    pos_flat = rows * _N + jp
    neg_flat = (rows[:, None] * _N + jn).reshape(-1)

    partials = _gather_loss(
        sim_i.reshape(_B * _N),
        sim_f.reshape(_B * _N),
        pos_flat, neg_flat,
    )

    out = pl.pallas_call(
        _combine_body,
        out_shape=jax.ShapeDtypeStruct((1, 1), jnp.float32),
    )(partials.reshape(_NW * 4, 16))
    return out[0, 0]


# speedup vs baseline: 1.6649x; 1.0219x over previous
import functools

import jax
import jax.numpy as jnp
from jax import lax
from jax.experimental import pallas as pl
from jax.experimental.pallas import tpu as pltpu
from jax.experimental.pallas import tpu_sc as plsc

_B, _N = 1024, 32768
_MARGIN = 0.1
_TOPK = 10
_BB = 32

_NC, _NS = 2, 16
_NW = _NC * _NS
_RPW = _B // _NW
_D = 128
_SEG = _N // _D


def _select_body(tgt_ref, simf_ref, noise_ref, idx_ref):
    tgt = tgt_ref[...]
    simf = simf_ref[...]
    noise = noise_ref[...]

    neg_inf = jnp.float32(-jnp.inf)
    col = jax.lax.broadcasted_iota(jnp.int32, (_BB, _N), 1)
    lanek = jax.lax.broadcasted_iota(jnp.int32, (_BB, 16), 1)
    acc = jnp.zeros((_BB, 16), jnp.int32)

    ub = lax.shift_right_logical(noise, 9)
    z = jnp.where(tgt > 0, ub, jnp.int32(-1))
    zmax = jnp.max(z, axis=1)
    jp = jnp.min(jnp.where(z == zmax[:, None], col, jnp.int32(_N)), axis=1)
    acc = jnp.where(lanek == _TOPK, jp[:, None], acc)

    s = jnp.where(tgt == 0, simf, jnp.float32(-50.0))
    for k in range(_TOPK):
        m = jnp.max(s, axis=1)
        jn = jnp.min(jnp.where(s == m[:, None], col, jnp.int32(_N)), axis=1)
        acc = jnp.where(lanek == k, jn[:, None], acc)
        s = jnp.where(col == jn[:, None], neg_inf, s)

    idx_ref[...] = acc


_sc_mesh = plsc.VectorSubcoreMesh(core_axis_name="c", subcore_axis_name="s")


@functools.partial(
    pl.kernel,
    mesh=_sc_mesh,
    out_type=jax.ShapeDtypeStruct((_NW * 64,), jnp.float32),
    scratch_types=[
        pltpu.VMEM((_RPW,), jnp.int32),
        pltpu.VMEM((_RPW * _TOPK,), jnp.int32),
        pltpu.VMEM((_RPW,), jnp.float32),
        pltpu.VMEM((_RPW,), jnp.float32),
        pltpu.VMEM((_RPW * _TOPK,), jnp.float32),
        pltpu.VMEM((_RPW * _TOPK,), jnp.float32),
        pltpu.VMEM((64,), jnp.float32),
        pltpu.SemaphoreType.DMA,
    ],
)
def _gather_loss(ti_hbm, tf_hbm, pidx_hbm, nidx_hbm, out_hbm, pidx_v, nidx_v,
                 pi_v, pf_v, ni_v, nf_v, acc_v, sem):
    wid = lax.axis_index("s") * _NC + lax.axis_index("c")
    pbase = wid * _RPW
    nbase = wid * (_RPW * _TOPK)
    pltpu.sync_copy(pidx_hbm.at[pl.ds(pbase, _RPW)], pidx_v)
    pltpu.sync_copy(nidx_hbm.at[pl.ds(nbase, _RPW * _TOPK)], nidx_v)
    pltpu.async_copy(ti_hbm.at[pidx_v], pi_v, sem).wait()
    pltpu.async_copy(tf_hbm.at[pidx_v], pf_v, sem).wait()
    pltpu.async_copy(ti_hbm.at[nidx_v], ni_v, sem).wait()
    pltpu.async_copy(tf_hbm.at[nidx_v], nf_v, sem).wait()

    margin = jnp.full((16,), _MARGIN, jnp.float32)
    zero = jnp.zeros((16,), jnp.float32)
    one = jnp.ones((16,), jnp.float32)

    sp = zero
    cp = zero
    for c in range(_RPW // 16):
        si = pi_v[pl.ds(c * 16, 16)]
        sf = pf_v[pl.ds(c * 16, 16)]
        t = jnp.maximum(si - sf + margin, zero)
        sp = sp + t
        cp = cp + jnp.where(t > zero, one, zero)

    sn = zero
    cn = zero
    for c in range(_RPW * _TOPK // 16):
        si = ni_v[pl.ds(c * 16, 16)]
        sf = nf_v[pl.ds(c * 16, 16)]
        t = jnp.maximum(sf - si + margin, zero)
        sn = sn + t
        cn = cn + jnp.where(t > zero, one, zero)

    acc_v[pl.ds(0, 16)] = sp
    acc_v[pl.ds(16, 16)] = cp
    acc_v[pl.ds(32, 16)] = sn
    acc_v[pl.ds(48, 16)] = cn
    pltpu.sync_copy(acc_v, out_hbm.at[pl.ds(wid * 64, 64)])


def _combine_body(p_ref, out_ref):
    p = p_ref[...]
    kind = jax.lax.broadcasted_iota(jnp.int32, (_NW * 4, 16), 0) % 4
    zero = jnp.float32(0.0)
    sp = jnp.sum(jnp.where(kind == 0, p, zero))
    cp = jnp.sum(jnp.where(kind == 1, p, zero))
    sn = jnp.sum(jnp.where(kind == 2, p, zero))
    cn = jnp.sum(jnp.where(kind == 3, p, zero))
    lp = jnp.where(sp == zero, zero, sp / jnp.maximum(cp, 1.0))
    ln = jnp.where(sn == zero, zero, sn / jnp.maximum(cn, 1.0))
    out_ref[...] = ((lp + ln) * 0.5).reshape(1, 1)


def kernel(sim_i, sim_f, target):
    noise = jax.lax.bitcast_convert_type(
        jax.random.bits(jax.random.key(42), (_B, _N), jnp.uint32), jnp.int32)

    spec = pl.BlockSpec((_BB, _N), lambda i: (i, 0))
    idx = pl.pallas_call(
        _select_body,
        grid=(_B // _BB,),
        in_specs=[spec, spec, spec],
        out_specs=pl.BlockSpec((_BB, 16), lambda i: (i, 0)),
        out_shape=jax.ShapeDtypeStruct((_B, 16), jnp.int32),
    )(target, sim_f, noise)

    jn = idx[:, :_TOPK]
    jp = idx[:, _TOPK]
    rows = jnp.arange(_B, dtype=jnp.int32)
    pos_flat = rows * _N + jp
    neg_flat = (rows[:, None] * _N + jn).reshape(-1)

    partials = _gather_loss(
        sim_i.reshape(_B * _N),
        sim_f.reshape(_B * _N),
        pos_flat, neg_flat,
    )

    out = pl.pallas_call(
        _combine_body,
        out_shape=jax.ShapeDtypeStruct((1, 1), jnp.float32),
    )(partials.reshape(_NW * 4, 16))
    return out[0, 0]
